# baseline (speedup 1.0000x reference)
import functools

import jax
import jax.numpy as jnp
from jax import lax
from jax.experimental import pallas as pl
from jax.experimental.pallas import tpu as pltpu
from jax.experimental.pallas import tpu_sc as plsc

_N, _D, _E = 10000, 128, 320000
_CHUNK = 64
_NBUF = 4
_ROWS_PER_TILE = 640


def _sc_scatter_add(x, src, dst):
    info = plsc.get_sparse_core_info()
    nc, ns = info.num_cores, info.num_subcores
    nw = nc * ns
    n_chunks = _E // _CHUNK
    assert _E % _CHUNK == 0
    n_pad = _ROWS_PER_TILE * ns
    zp = _ROWS_PER_TILE // _CHUNK
    assert _ROWS_PER_TILE % _CHUNK == 0
    tail_rows = _N - (ns - 1) * _ROWS_PER_TILE
    assert tail_rows > 0 and (ns - 1) * _ROWS_PER_TILE % 8 == 0

    mesh = plsc.VectorSubcoreMesh(core_axis_name="c", subcore_axis_name="s")

    @functools.partial(
        pl.kernel,
        out_type=jax.ShapeDtypeStruct((nc, _N, _D), jnp.float32),
        mesh=mesh,
        scratch_types=[
            pltpu.MemorySpace.VMEM_SHARED((n_pad, _D), jnp.float32),
            [pltpu.MemorySpace.VMEM((_CHUNK,), jnp.int32)] * _NBUF,
            [pltpu.MemorySpace.VMEM((_CHUNK,), jnp.int32)] * _NBUF,
            [pltpu.MemorySpace.VMEM((_CHUNK, _D), jnp.float32)] * _NBUF,
            [pltpu.SemaphoreType.DMA] * _NBUF,
            [pltpu.SemaphoreType.DMA] * _NBUF,
            [pltpu.SemaphoreType.DMA] * _NBUF,
            [pltpu.SemaphoreType.DMA] * _NBUF,
        ],
    )
    def k(x_hbm, src_hbm, dst_hbm, out_hbm, agg_sh, src_v, dst_v, rows,
          sem_is, sem_id, sem_g, sem_s):
        c = lax.axis_index("c")
        s = lax.axis_index("s")
        wid = s * nc + c

        def zrow(i, _):
            def zlane(j, _):
                rows[0][i, pl.ds(j * 16, 16)] = jnp.zeros((16,), jnp.float32)
                return 0

            lax.fori_loop(0, _D // 16, zlane, 0)
            return 0

        lax.fori_loop(0, _CHUNK, zrow, 0)

        for p in range(zp):
            pltpu.sync_copy(
                rows[0],
                agg_sh.at[pl.ds(s * _ROWS_PER_TILE + p * _CHUNK, _CHUNK)],
            )

        plsc.subcore_barrier()

        base_count = n_chunks // nw
        extra = n_chunks % nw
        assert base_count % _NBUF == 0

        def body(i, _):
            idx_d = []
            for b in range(_NBUF):
                base = ((i * _NBUF + b) * nw + wid) * _CHUNK
                idx_d.append((
                    pltpu.async_copy(src_hbm.at[pl.ds(base, _CHUNK)],
                                     src_v[b], sem_is[b]),
                    pltpu.async_copy(dst_hbm.at[pl.ds(base, _CHUNK)],
                                     dst_v[b], sem_id[b]),
                ))
            g_d = []
            for b in range(_NBUF):
                idx_d[b][0].wait()
                g_d.append(pltpu.async_copy(x_hbm.at[src_v[b]], rows[b],
                                            sem_g[b]))
            s_d = []
            for b in range(_NBUF):
                g_d[b].wait()
                idx_d[b][1].wait()
                s_d.append(pltpu.async_copy(rows[b], agg_sh.at[dst_v[b]],
                                            sem_s[b], add=True))
            for b in range(_NBUF):
                s_d[b].wait()
            return 0

        lax.fori_loop(0, base_count // _NBUF, body, 0)

        @pl.when(wid < extra)
        def _():
            base = (base_count * nw + wid) * _CHUNK
            pltpu.sync_copy(src_hbm.at[pl.ds(base, _CHUNK)], src_v[0])
            pltpu.sync_copy(dst_hbm.at[pl.ds(base, _CHUNK)], dst_v[0])
            pltpu.async_copy(x_hbm.at[src_v[0]], rows[0], sem_g[0]).wait()
            pltpu.sync_copy(rows[0], agg_sh.at[dst_v[0]], add=True)

        plsc.subcore_barrier()

        @pl.when(s < ns - 1)
        def _():
            pltpu.sync_copy(
                agg_sh.at[pl.ds(s * _ROWS_PER_TILE, _ROWS_PER_TILE)],
                out_hbm.at[c, pl.ds(s * _ROWS_PER_TILE, _ROWS_PER_TILE)],
            )

        @pl.when(s == ns - 1)
        def _():
            pltpu.sync_copy(
                agg_sh.at[pl.ds((ns - 1) * _ROWS_PER_TILE, tail_rows)],
                out_hbm.at[c, pl.ds((ns - 1) * _ROWS_PER_TILE, tail_rows)],
            )

    return k(x, src, dst)


_BLK = 1000
_NBLK = _N // _BLK


def _mlp_body(x_ref, p_ref, w1_ref, b1_ref, w2_ref, b2_ref, wg_ref, bg_ref,
              wa_ref, ba_ref, h_ref, acol_ref):
    dn = (((1,), (1,)), ((), ()))
    xa = x_ref[...] + p_ref[0] + p_ref[1]
    h1 = lax.dot_general(xa, w1_ref[...], dn, preferred_element_type=jnp.float32)
    h1 = jnp.maximum(h1 + b1_ref[...], 0.0)
    h = lax.dot_general(h1, w2_ref[...], dn, preferred_element_type=jnp.float32)
    h = h + b2_ref[...]
    h_ref[...] = h
    ga = lax.dot_general(h, wg_ref[...], dn, preferred_element_type=jnp.float32)
    ga = jnp.tanh(ga + bg_ref[...])
    acol_ref[...] = lax.dot_general(ga, wa_ref[...], dn,
                                    preferred_element_type=jnp.float32)


def _softmax_body(al_ref, a_ref):
    al = al_ref[...]
    e = jnp.exp(al - jnp.max(al))
    a_ref[...] = e / jnp.sum(e)


def kernel(x, edge_index, W1, b1, W2, b2, Wg, bg, Wa, ba):
    src = edge_index[0]
    dst = edge_index[1]
    parts = _sc_scatter_add(x, src, dst)
    wspec = pl.BlockSpec((_D, _D), lambda i: (0, 0))
    bspec = pl.BlockSpec((1, _D), lambda i: (0, 0))
    h, acol = pl.pallas_call(
        _mlp_body,
        grid=(_NBLK,),
        in_specs=[
            pl.BlockSpec((_BLK, _D), lambda i: (i, 0)),
            pl.BlockSpec((2, _BLK, _D), lambda i: (0, i, 0)),
            wspec, bspec, wspec, bspec, wspec, bspec,
            bspec, pl.BlockSpec((1, 1), lambda i: (0, 0)),
        ],
        out_specs=[
            pl.BlockSpec((_BLK, _D), lambda i: (i, 0)),
            pl.BlockSpec((_BLK, 1), lambda i: (i, 0)),
        ],
        out_shape=[
            jax.ShapeDtypeStruct((_N, _D), jnp.float32),
            jax.ShapeDtypeStruct((_N, 1), jnp.float32),
        ],
    )(x, parts, W1, b1.reshape(1, -1), W2, b2.reshape(1, -1),
      Wg, bg.reshape(1, -1), Wa, ba.reshape(1, 1))
    a = pl.pallas_call(
        _softmax_body,
        out_shape=jax.ShapeDtypeStruct((_N, 1), jnp.float32),
    )(acol)
    return h, a[:, 0]


# speedup vs baseline: 1.0821x; 1.0821x over previous
import functools

import jax
import jax.numpy as jnp
from jax import lax
from jax.experimental import pallas as pl
from jax.experimental.pallas import tpu as pltpu
from jax.experimental.pallas import tpu_sc as plsc

_N, _D, _E = 10000, 128, 320000
_CHUNK = 64
_NBUF = 4
_ROWS_PER_TILE = 640


def _sc_scatter_add(x, src, dst):
    info = plsc.get_sparse_core_info()
    nc, ns = info.num_cores, info.num_subcores
    nw = nc * ns
    n_chunks = _E // _CHUNK
    assert _E % _CHUNK == 0
    n_pad = _ROWS_PER_TILE * ns
    zp = _ROWS_PER_TILE // _CHUNK
    assert _ROWS_PER_TILE % _CHUNK == 0
    tail_rows = _N - (ns - 1) * _ROWS_PER_TILE
    assert tail_rows > 0 and (ns - 1) * _ROWS_PER_TILE % 8 == 0

    mesh = plsc.VectorSubcoreMesh(core_axis_name="c", subcore_axis_name="s")

    @functools.partial(
        pl.kernel,
        out_type=jax.ShapeDtypeStruct((nc, _N, _D), jnp.float32),
        mesh=mesh,
        scratch_types=[
            pltpu.MemorySpace.VMEM_SHARED((n_pad, _D), jnp.float32),
            [pltpu.MemorySpace.VMEM((_CHUNK,), jnp.int32)] * _NBUF,
            [pltpu.MemorySpace.VMEM((_CHUNK,), jnp.int32)] * _NBUF,
            [pltpu.MemorySpace.VMEM((_CHUNK, _D), jnp.float32)] * _NBUF,
            [pltpu.SemaphoreType.DMA] * _NBUF,
            [pltpu.SemaphoreType.DMA] * _NBUF,
            [pltpu.SemaphoreType.DMA] * _NBUF,
            [pltpu.SemaphoreType.DMA] * _NBUF,
        ],
    )
    def k(x_hbm, src_hbm, dst_hbm, out_hbm, agg_sh, src_v, dst_v, rows,
          sem_is, sem_id, sem_g, sem_s):
        c = lax.axis_index("c")
        s = lax.axis_index("s")
        wid = s * nc + c

        def zrow(i, _):
            def zlane(j, _):
                rows[0][i, pl.ds(j * 16, 16)] = jnp.zeros((16,), jnp.float32)
                return 0

            lax.fori_loop(0, _D // 16, zlane, 0)
            return 0

        lax.fori_loop(0, _CHUNK, zrow, 0)

        for p in range(zp):
            pltpu.sync_copy(
                rows[0],
                agg_sh.at[pl.ds(s * _ROWS_PER_TILE + p * _CHUNK, _CHUNK)],
            )

        plsc.subcore_barrier()

        base_count = n_chunks // nw
        extra = n_chunks % nw
        assert base_count % _NBUF == 0

        def body(i, _):
            idx_d = []
            for b in range(_NBUF):
                base = ((i * _NBUF + b) * nw + wid) * _CHUNK
                idx_d.append((
                    pltpu.async_copy(src_hbm.at[pl.ds(base, _CHUNK)],
                                     src_v[b], sem_is[b]),
                    pltpu.async_copy(dst_hbm.at[pl.ds(base, _CHUNK)],
                                     dst_v[b], sem_id[b]),
                ))
            g_d = []
            for b in range(_NBUF):
                idx_d[b][0].wait()
                g_d.append(pltpu.async_copy(x_hbm.at[src_v[b]], rows[b],
                                            sem_g[b]))
            s_d = []
            for b in range(_NBUF):
                g_d[b].wait()
                idx_d[b][1].wait()
                s_d.append(pltpu.async_copy(rows[b], agg_sh.at[dst_v[b]],
                                            sem_s[b], add=True))
            for b in range(_NBUF):
                s_d[b].wait()
            return 0

        lax.fori_loop(0, base_count // _NBUF, body, 0)

        @pl.when(wid < extra)
        def _():
            base = (base_count * nw + wid) * _CHUNK
            pltpu.sync_copy(src_hbm.at[pl.ds(base, _CHUNK)], src_v[0])
            pltpu.sync_copy(dst_hbm.at[pl.ds(base, _CHUNK)], dst_v[0])
            pltpu.async_copy(x_hbm.at[src_v[0]], rows[0], sem_g[0]).wait()
            pltpu.sync_copy(rows[0], agg_sh.at[dst_v[0]], add=True)

        plsc.subcore_barrier()

        @pl.when(s < ns - 1)
        def _():
            pltpu.sync_copy(
                agg_sh.at[pl.ds(s * _ROWS_PER_TILE, _ROWS_PER_TILE)],
                out_hbm.at[c, pl.ds(s * _ROWS_PER_TILE, _ROWS_PER_TILE)],
            )

        @pl.when(s == ns - 1)
        def _():
            pltpu.sync_copy(
                agg_sh.at[pl.ds((ns - 1) * _ROWS_PER_TILE, tail_rows)],
                out_hbm.at[c, pl.ds((ns - 1) * _ROWS_PER_TILE, tail_rows)],
            )

    return k(x, src, dst)


def _mlp_body(x_ref, p_ref, w1_ref, b1_ref, w2_ref, b2_ref, wg_ref, bg_ref,
              wa_ref, ba_ref, h_ref, a_ref):
    dn = (((1,), (1,)), ((), ()))
    xa = x_ref[...] + p_ref[0] + p_ref[1]
    h1 = lax.dot_general(xa, w1_ref[...], dn, preferred_element_type=jnp.float32)
    h1 = jnp.maximum(h1 + b1_ref[...], 0.0)
    h = lax.dot_general(h1, w2_ref[...], dn, preferred_element_type=jnp.float32)
    h = h + b2_ref[...]
    h_ref[...] = h
    ga = lax.dot_general(h, wg_ref[...], dn, preferred_element_type=jnp.float32)
    ga = jnp.tanh(ga + bg_ref[...])
    alpha = lax.dot_general(wa_ref[...], ga, dn, preferred_element_type=jnp.float32)
    e = jnp.exp(alpha - jnp.max(alpha))
    a_ref[...] = e / jnp.sum(e)


def kernel(x, edge_index, W1, b1, W2, b2, Wg, bg, Wa, ba):
    src = edge_index[0]
    dst = edge_index[1]
    parts = _sc_scatter_add(x, src, dst)
    h, a = pl.pallas_call(
        _mlp_body,
        out_shape=[
            jax.ShapeDtypeStruct((_N, _D), jnp.float32),
            jax.ShapeDtypeStruct((1, _N), jnp.float32),
        ],
    )(x, parts, W1, b1.reshape(1, -1), W2, b2.reshape(1, -1),
      Wg, bg.reshape(1, -1), Wa, ba.reshape(1, 1))
    return h, a[0]
